# baseline (speedup 1.0000x reference)
import jax
import jax.numpy as jnp
from jax.experimental import pallas as pl


def _copy_kernel(x_ref, o_ref):
    o_ref[...] = x_ref[...]


def kernel(inputs_embeds, embed_table):
    del embed_table
    return pl.pallas_call(
        _copy_kernel,
        out_shape=jax.ShapeDtypeStruct(inputs_embeds.shape, inputs_embeds.dtype),
    )(inputs_embeds)
